# Initial kernel scaffold; baseline (speedup 1.0000x reference)
import jax
import jax.numpy as jnp
from jax.experimental import pallas as pl


def kernel(prop_atom_in_ring, prop_atom_charge, prop_atom_hybridization, prop_atom_chirality, W_ring, W_charge, W_hyb, W_chir):
    raise NotImplementedError("write your pallas kernel here")


# speedup vs baseline: 3.0044x; 3.0044x over previous
import functools

import jax
import jax.numpy as jnp
from jax import lax
from jax.experimental import pallas as pl
from jax.experimental.pallas import tpu as pltpu
from jax.experimental.pallas import tpu_sc as plsc

B, L = 1024, 200
BL = B * L
PER_PROP = 32
EMB = 4 * PER_PROP
NC, NS = 2, 16
NW = NC * NS
PER_W = BL // NW
CHUNK = 400
NCHUNK = PER_W // CHUNK
NGROUP = CHUNK // 16
OFFS = (0, 3, 7, 16)


def _body(i0, i1, i2, i3, wall, out, wtab, iv0, iv1, iv2, iv3, obuf):
    wid = lax.axis_index("s") * NC + lax.axis_index("c")
    pltpu.sync_copy(wall, wtab)
    lanes = lax.iota(jnp.int32, 16)

    def chunk_body(c, carry):
        base = wid * PER_W + c * CHUNK
        pltpu.sync_copy(i0.at[pl.ds(base, CHUNK)], iv0)
        pltpu.sync_copy(i1.at[pl.ds(base, CHUNK)], iv1)
        pltpu.sync_copy(i2.at[pl.ds(base, CHUNK)], iv2)
        pltpu.sync_copy(i3.at[pl.ds(base, CHUNK)], iv3)

        def group(s, carry2):
            rows16 = s * 16 + lanes
            for j, (ivj, off) in enumerate(
                ((iv0, OFFS[0]), (iv1, OFFS[1]), (iv2, OFFS[2]), (iv3, OFFS[3]))
            ):
                vj = plsc.load_gather(ivj, [rows16]) + off
                src = vj * PER_PROP
                dst = rows16 * EMB + j * PER_PROP
                for col in range(PER_PROP):
                    vals = plsc.load_gather(wtab, [src + col])
                    plsc.store_scatter(obuf, [dst + col], vals)
            return carry2

        lax.fori_loop(0, NGROUP, group, 0)
        pltpu.sync_copy(obuf, out.at[pl.ds(base * EMB, CHUNK * EMB)])
        return carry

    lax.fori_loop(0, NCHUNK, chunk_body, 0)


@jax.jit
def kernel(
    prop_atom_in_ring,
    prop_atom_charge,
    prop_atom_hybridization,
    prop_atom_chirality,
    W_ring,
    W_charge,
    W_hyb,
    W_chir,
):
    wall = jnp.concatenate([W_ring, W_charge, W_hyb, W_chir], axis=0).reshape(-1)
    mesh = plsc.VectorSubcoreMesh(core_axis_name="c", subcore_axis_name="s")
    k = functools.partial(
        pl.kernel,
        mesh=mesh,
        compiler_params=pltpu.CompilerParams(needs_layout_passes=False),
        out_type=jax.ShapeDtypeStruct((BL * EMB,), jnp.float32),
        scratch_types=[
            pltpu.VMEM((21 * PER_PROP,), jnp.float32),
            pltpu.VMEM((CHUNK,), jnp.int32),
            pltpu.VMEM((CHUNK,), jnp.int32),
            pltpu.VMEM((CHUNK,), jnp.int32),
            pltpu.VMEM((CHUNK,), jnp.int32),
            pltpu.VMEM((CHUNK * EMB,), jnp.float32),
        ],
    )(_body)
    out = k(
        prop_atom_in_ring.reshape(BL),
        prop_atom_charge.reshape(BL),
        prop_atom_hybridization.reshape(BL),
        prop_atom_chirality.reshape(BL),
        wall,
    )
    return out.reshape(B, L, EMB)


# speedup vs baseline: 4.1406x; 1.3782x over previous
import functools

import jax
import jax.numpy as jnp
from jax import lax
from jax.experimental import pallas as pl
from jax.experimental.pallas import tpu as pltpu
from jax.experimental.pallas import tpu_sc as plsc

B, L = 1024, 200
BL = B * L
PER_PROP = 32
EMB = 4 * PER_PROP
NC, NS = 2, 16
NW = NC * NS
PER_W = BL // NW
CHUNK = 400
NCHUNK = PER_W // CHUNK
NGROUP = CHUNK // 16
OFFS = (0, 3, 7, 16)


def _body(i0, i1, i2, i3, wall, out, wtab, iv0, iv1, iv2, iv3, obuf):
    wid = lax.axis_index("s") * NC + lax.axis_index("c")
    pltpu.sync_copy(wall, wtab)
    lanes = lax.iota(jnp.int32, 16)

    def chunk_body(c, carry):
        base = wid * PER_W + c * CHUNK
        pltpu.sync_copy(i0.at[pl.ds(base, CHUNK)], iv0)
        pltpu.sync_copy(i1.at[pl.ds(base, CHUNK)], iv1)
        pltpu.sync_copy(i2.at[pl.ds(base, CHUNK)], iv2)
        pltpu.sync_copy(i3.at[pl.ds(base, CHUNK)], iv3)

        def group(s, carry2):
            rows16 = s * 16 + lanes
            srcs = []
            dsts = []
            for j, (ivj, off) in enumerate(
                ((iv0, OFFS[0]), (iv1, OFFS[1]), (iv2, OFFS[2]), (iv3, OFFS[3]))
            ):
                vj = plsc.load_gather(ivj, [rows16]) + off
                srcs.append(vj * PER_PROP)
                dsts.append(rows16 * EMB + j * PER_PROP)

            def gath(p):
                j, col = divmod(p, PER_PROP)
                return plsc.load_gather(wtab, [srcs[j] + col])

            def scat(p, vals):
                j, col = divmod(p, PER_PROP)
                plsc.store_scatter(obuf, [dsts[j] + col], vals)

            BK = 16
            NB = EMB // BK
            prev = [gath(p) for p in range(BK)]
            cur = [gath(p) for p in range(BK, 2 * BK)]
            for b in range(2, NB):
                nxt = [gath(p) for p in range(b * BK, (b + 1) * BK)]
                for k in range(BK):
                    scat((b - 2) * BK + k, prev[k])
                prev, cur = cur, nxt
            for k in range(BK):
                scat((NB - 2) * BK + k, prev[k])
            for k in range(BK):
                scat((NB - 1) * BK + k, cur[k])
            return carry2

        lax.fori_loop(0, NGROUP, group, 0)
        pltpu.sync_copy(obuf, out.at[pl.ds(base * EMB, CHUNK * EMB)])
        return carry

    lax.fori_loop(0, NCHUNK, chunk_body, 0)


@jax.jit
def kernel(
    prop_atom_in_ring,
    prop_atom_charge,
    prop_atom_hybridization,
    prop_atom_chirality,
    W_ring,
    W_charge,
    W_hyb,
    W_chir,
):
    wall = jnp.concatenate([W_ring, W_charge, W_hyb, W_chir], axis=0).reshape(-1)
    mesh = plsc.VectorSubcoreMesh(core_axis_name="c", subcore_axis_name="s")
    k = functools.partial(
        pl.kernel,
        mesh=mesh,
        compiler_params=pltpu.CompilerParams(needs_layout_passes=False),
        out_type=jax.ShapeDtypeStruct((BL * EMB,), jnp.float32),
        scratch_types=[
            pltpu.VMEM((21 * PER_PROP,), jnp.float32),
            pltpu.VMEM((CHUNK,), jnp.int32),
            pltpu.VMEM((CHUNK,), jnp.int32),
            pltpu.VMEM((CHUNK,), jnp.int32),
            pltpu.VMEM((CHUNK,), jnp.int32),
            pltpu.VMEM((CHUNK * EMB,), jnp.float32),
        ],
    )(_body)
    out = k(
        prop_atom_in_ring.reshape(BL),
        prop_atom_charge.reshape(BL),
        prop_atom_hybridization.reshape(BL),
        prop_atom_chirality.reshape(BL),
        wall,
    )
    return out.reshape(B, L, EMB)


# speedup vs baseline: 4.4790x; 1.0817x over previous
import functools

import jax
import jax.numpy as jnp
from jax import lax
from jax.experimental import pallas as pl
from jax.experimental.pallas import tpu as pltpu
from jax.experimental.pallas import tpu_sc as plsc

B, L = 1024, 200
BL = B * L
PER_PROP = 32
EMB = 4 * PER_PROP
NC, NS = 2, 16
NW = NC * NS
PER_W = BL // NW
CHUNK = 400
NOUTER = PER_W // (2 * CHUNK)
NGROUP = CHUNK // 16
OFFS = (0, 3, 7, 16)


def _body(i0, i1, i2, i3, wall, out, wtab, iv0, iv1, iv2, iv3, obufA, obufB, semA, semB):
    wid = lax.axis_index("s") * NC + lax.axis_index("c")
    span = wid * PER_W
    pltpu.sync_copy(wall, wtab)
    pltpu.sync_copy(i0.at[pl.ds(span, PER_W)], iv0)
    pltpu.sync_copy(i1.at[pl.ds(span, PER_W)], iv1)
    pltpu.sync_copy(i2.at[pl.ds(span, PER_W)], iv2)
    pltpu.sync_copy(i3.at[pl.ds(span, PER_W)], iv3)
    lanes = lax.iota(jnp.int32, 16)

    def make_chunk(obuf, sem):
        def run_chunk(t, c):
            @pl.when(t > 0)
            def _():
                pltpu.make_async_copy(
                    obuf, out.at[pl.ds((span + c * CHUNK) * EMB, CHUNK * EMB)], sem
                ).wait()

            def group(s, carry2):
                loc16 = s * 16 + lanes
                abs16 = c * CHUNK + loc16
                srcs = []
                dsts = []
                for j, (ivj, off) in enumerate(
                    ((iv0, OFFS[0]), (iv1, OFFS[1]), (iv2, OFFS[2]), (iv3, OFFS[3]))
                ):
                    vj = plsc.load_gather(ivj, [abs16]) + off
                    srcs.append(vj * PER_PROP)
                    dsts.append(loc16 * EMB + j * PER_PROP)

                def gath(p):
                    j, col = divmod(p, PER_PROP)
                    return plsc.load_gather(wtab, [srcs[j] + col])

                def scat(p, vals):
                    j, col = divmod(p, PER_PROP)
                    plsc.store_scatter(obuf, [dsts[j] + col], vals)

                BK = 16
                NB = EMB // BK
                prev = [gath(p) for p in range(BK)]
                cur = [gath(p) for p in range(BK, 2 * BK)]
                for b in range(2, NB):
                    nxt = [gath(p) for p in range(b * BK, (b + 1) * BK)]
                    for k in range(BK):
                        scat((b - 2) * BK + k, prev[k])
                    prev, cur = cur, nxt
                for k in range(BK):
                    scat((NB - 2) * BK + k, prev[k])
                for k in range(BK):
                    scat((NB - 1) * BK + k, cur[k])
                return carry2

            lax.fori_loop(0, NGROUP, group, 0)
            pltpu.make_async_copy(
                obuf, out.at[pl.ds((span + c * CHUNK) * EMB, CHUNK * EMB)], sem
            ).start()

        return run_chunk

    chunkA = make_chunk(obufA, semA)
    chunkB = make_chunk(obufB, semB)

    def outer(t, carry):
        chunkA(t, 2 * t)
        chunkB(t, 2 * t + 1)
        return carry

    lax.fori_loop(0, NOUTER, outer, 0)
    pltpu.make_async_copy(obufA, out.at[pl.ds(span * EMB, CHUNK * EMB)], semA).wait()
    pltpu.make_async_copy(obufB, out.at[pl.ds(span * EMB, CHUNK * EMB)], semB).wait()


@jax.jit
def kernel(
    prop_atom_in_ring,
    prop_atom_charge,
    prop_atom_hybridization,
    prop_atom_chirality,
    W_ring,
    W_charge,
    W_hyb,
    W_chir,
):
    wall = jnp.concatenate([W_ring, W_charge, W_hyb, W_chir], axis=0).reshape(-1)
    mesh = plsc.VectorSubcoreMesh(core_axis_name="c", subcore_axis_name="s")
    k = functools.partial(
        pl.kernel,
        mesh=mesh,
        compiler_params=pltpu.CompilerParams(needs_layout_passes=False),
        out_type=jax.ShapeDtypeStruct((BL * EMB,), jnp.float32),
        scratch_types=[
            pltpu.VMEM((21 * PER_PROP,), jnp.float32),
            pltpu.VMEM((PER_W,), jnp.int32),
            pltpu.VMEM((PER_W,), jnp.int32),
            pltpu.VMEM((PER_W,), jnp.int32),
            pltpu.VMEM((PER_W,), jnp.int32),
            pltpu.VMEM((CHUNK * EMB,), jnp.float32),
            pltpu.VMEM((CHUNK * EMB,), jnp.float32),
            pltpu.SemaphoreType.DMA,
            pltpu.SemaphoreType.DMA,
        ],
    )(_body)
    out = k(
        prop_atom_in_ring.reshape(BL),
        prop_atom_charge.reshape(BL),
        prop_atom_hybridization.reshape(BL),
        prop_atom_chirality.reshape(BL),
        wall,
    )
    return out.reshape(B, L, EMB)


# speedup vs baseline: 30.5833x; 6.8282x over previous
import functools

import jax
import jax.numpy as jnp
from jax import lax
from jax.experimental import pallas as pl
from jax.experimental.pallas import tpu as pltpu
from jax.experimental.pallas import tpu_sc as plsc

B, L = 1024, 200
BL = B * L
PER_PROP = 32
EMB = 4 * PER_PROP
NC, NS = 2, 16
NW = NC * NS
PER_W = BL // NW
CHUNK = 320
NOUTER = PER_W // (2 * CHUNK)
NITER = CHUNK // 16
OFFS = (0, 3, 7, 16)


def _body(i0, i1, i2, i3, wall, out, wtab, iv0, iv1, iv2, iv3, obufA, obufB, semA, semB):
    wid = lax.axis_index("s") * NC + lax.axis_index("c")
    span = wid * PER_W
    pltpu.sync_copy(wall, wtab)
    pltpu.sync_copy(i0.at[pl.ds(span, PER_W)], iv0)
    pltpu.sync_copy(i1.at[pl.ds(span, PER_W)], iv1)
    pltpu.sync_copy(i2.at[pl.ds(span, PER_W)], iv2)
    pltpu.sync_copy(i3.at[pl.ds(span, PER_W)], iv3)
    ivs = (iv0, iv1, iv2, iv3)

    def make_chunk(obuf, sem):
        def run_chunk(t, c):
            @pl.when(t > 0)
            def _():
                pltpu.make_async_copy(
                    obuf, out.at[pl.ds((span + c * CHUNK) * EMB, CHUNK * EMB)], sem
                ).wait()

            def iteration(it, carry):
                ebase = c * CHUNK + it * 16
                obase = pl.multiple_of(it * (16 * EMB), 16 * EMB)
                vjs = [ivs[j][pl.ds(ebase, 16)] for j in range(4)]

                def load_block(blk):
                    vals = []
                    for k in range(2):
                        e = blk * 2 + k
                        for j in range(4):
                            rb = pl.multiple_of(vjs[j][e] * PER_PROP, PER_PROP)
                            soff = OFFS[j] * PER_PROP
                            for h in range(2):
                                vals.append(wtab[pl.ds(rb + soff + 16 * h, 16)])
                    return vals

                def store_block(blk, vals):
                    for k in range(2):
                        e = blk * 2 + k
                        for j in range(4):
                            for h in range(2):
                                obuf[
                                    pl.ds(obase + e * EMB + j * PER_PROP + 16 * h, 16)
                                ] = vals[(k * 4 + j) * 2 + h]

                prev = load_block(0)
                for blk in range(1, 8):
                    nxt = load_block(blk)
                    store_block(blk - 1, prev)
                    prev = nxt
                store_block(7, prev)
                return carry

            lax.fori_loop(0, NITER, iteration, 0)
            pltpu.make_async_copy(
                obuf, out.at[pl.ds((span + c * CHUNK) * EMB, CHUNK * EMB)], sem
            ).start()

        return run_chunk

    chunkA = make_chunk(obufA, semA)
    chunkB = make_chunk(obufB, semB)

    def outer(t, carry):
        chunkA(t, 2 * t)
        chunkB(t, 2 * t + 1)
        return carry

    lax.fori_loop(0, NOUTER, outer, 0)
    pltpu.make_async_copy(
        obufA, out.at[pl.ds(span * EMB, CHUNK * EMB)], semA
    ).wait()
    pltpu.make_async_copy(
        obufB, out.at[pl.ds(span * EMB, CHUNK * EMB)], semB
    ).wait()


@jax.jit
def kernel(
    prop_atom_in_ring,
    prop_atom_charge,
    prop_atom_hybridization,
    prop_atom_chirality,
    W_ring,
    W_charge,
    W_hyb,
    W_chir,
):
    wall = jnp.concatenate([W_ring, W_charge, W_hyb, W_chir], axis=0).reshape(-1)
    mesh = plsc.VectorSubcoreMesh(core_axis_name="c", subcore_axis_name="s")
    k = functools.partial(
        pl.kernel,
        mesh=mesh,
        compiler_params=pltpu.CompilerParams(
            needs_layout_passes=False, use_tc_tiling_on_sc=False
        ),
        out_type=jax.ShapeDtypeStruct((BL * EMB,), jnp.float32),
        scratch_types=[
            pltpu.VMEM((21 * PER_PROP,), jnp.float32),
            pltpu.VMEM((PER_W,), jnp.int32),
            pltpu.VMEM((PER_W,), jnp.int32),
            pltpu.VMEM((PER_W,), jnp.int32),
            pltpu.VMEM((PER_W,), jnp.int32),
            pltpu.VMEM((CHUNK * EMB,), jnp.float32),
            pltpu.VMEM((CHUNK * EMB,), jnp.float32),
            pltpu.SemaphoreType.DMA,
            pltpu.SemaphoreType.DMA,
        ],
    )(_body)
    out = k(
        prop_atom_in_ring.reshape(BL),
        prop_atom_charge.reshape(BL),
        prop_atom_hybridization.reshape(BL),
        prop_atom_chirality.reshape(BL),
        wall,
    )
    return out.reshape(B, L, EMB)
